# baseline (speedup 1.0000x reference)
import functools

import jax
import jax.numpy as jnp
from jax import lax
from jax.experimental import pallas as pl
from jax.experimental.pallas import tpu as pltpu
from jax.experimental.pallas import tpu_sc as plsc

_N = 10000
_E = 320000
_D = 128
_G = 16
_NPAD = 10240
_NC = 2
_NS = 16
_NW = _NC * _NS
_CH = 128
_CPW = 80
_EW = _CPW * _CH
_EPAD = _EW * _NW
_RPT = _NPAD // _NS
_RB = 512
_NBLK = _NPAD // _RB

def _msg_body_sc(g_hbm, src2_hbm, dst2_hbm, zeros_hbm, acc_out,
                 acc_sh, rows0, rows1, srcb, dstb,
                 semg0, semg1, sems0, sems1, semi):
    cx = lax.axis_index("c")
    sx = lax.axis_index("s")
    wid = sx * _NC + cx
    r0 = wid * _CPW
    row0 = sx * _RPT
    cp_src = pltpu.async_copy(src2_hbm.at[pl.ds(r0, _CPW // 2)], srcb, semi)
    cp_dst = pltpu.async_copy(dst2_hbm.at[pl.ds(r0, _CPW)], dstb, semi)
    pltpu.sync_copy(zeros_hbm, rows0)
    for j in range(_RPT // _CH):
        pltpu.sync_copy(rows0, acc_sh.at[pl.ds(row0 + j * _CH, _CH)])
    plsc.subcore_barrier()
    cp_src.wait()
    cp_dst.wait()

    rows = (rows0, rows1)
    semg = (semg0, semg1)
    sems = (sems0, sems1)
    cp_g = [None] * _CPW
    cp_s = [None] * _CPW
    half = _CPW // 2
    g_waited = [False] * _CPW
    for c in range(_CPW):
        if c == half:
            for d in (half - 1,):
                if not g_waited[d]:
                    cp_g[d].wait()
                    g_waited[d] = True
            pltpu.sync_copy(src2_hbm.at[pl.ds(r0 + half, half)], srcb)
        if c >= 2:
            cp_s[c - 2].wait()
        cb = c % 2
        cp_g[c] = pltpu.async_copy(
            g_hbm.at[srcb.at[c % half]], rows[cb], semg[cb])
        if c >= 1:
            d = c - 1
            if not g_waited[d]:
                cp_g[d].wait()
                g_waited[d] = True
            cp_s[d] = pltpu.async_copy(
                rows[d % 2], acc_sh.at[dstb.at[d]], sems[d % 2], add=True)
    d = _CPW - 1
    if not g_waited[d]:
        cp_g[d].wait()
        g_waited[d] = True
    cp_s[d] = pltpu.async_copy(
        rows[d % 2], acc_sh.at[dstb.at[d]], sems[d % 2], add=True)
    for d in range(_CPW - 2, _CPW):
        cp_s[d].wait()
    plsc.subcore_barrier()
    pltpu.sync_copy(acc_sh.at[pl.ds(row0, _RPT)],
                    acc_out.at[cx, pl.ds(row0, _RPT)])


def _deg_body_sc(dst2_hbm, ones_hbm, zeros_hbm, acc_out,
                 acc_sh, ones_v, z_v, dstb, sem0, sem1, sem2, sem3, semi):
    cx = lax.axis_index("c")
    sx = lax.axis_index("s")
    wid = sx * _NC + cx
    r0 = wid * _CPW
    row0 = sx * _RPT
    cp_dst = pltpu.async_copy(dst2_hbm.at[pl.ds(r0, _CPW)], dstb, semi)
    pltpu.sync_copy(ones_hbm, ones_v)
    pltpu.sync_copy(zeros_hbm, z_v)
    for j in range(_RPT // _CH):
        pltpu.sync_copy(z_v, acc_sh.at[pl.ds(row0 + j * _CH, _CH)])
    plsc.subcore_barrier()
    cp_dst.wait()
    sems = (sem0, sem1, sem2, sem3)
    cp_s = [None] * _CPW
    for c in range(_CPW):
        if c >= 4:
            cp_s[c - 4].wait()
        cp_s[c] = pltpu.async_copy(
            ones_v, acc_sh.at[dstb.at[c]], sems[c % 4], add=True)
    for c in range(_CPW - 4, _CPW):
        cp_s[c].wait()
    plsc.subcore_barrier()
    pltpu.sync_copy(acc_sh.at[pl.ds(row0, _RPT)],
                    acc_out.at[cx, pl.ds(row0, _RPT)])


def _dinv_from_hist(hist_ref):
    deg = hist_ref[0, :, 0:1] + hist_ref[1, :, 0:1] + 1.0
    return lax.rsqrt(deg)


def _hw_body(x_ref, w_ref, o_ref):
    o_ref[...] = jnp.dot(
        x_ref[...], w_ref[...], preferred_element_type=jnp.float32)


_hw_call = pl.pallas_call(
    _hw_body,
    grid=(_NBLK,),
    in_specs=[
        pl.BlockSpec((_RB, _D), lambda i: (i, 0)),
        pl.BlockSpec((_D, _D), lambda i: (0, 0)),
    ],
    out_specs=pl.BlockSpec((_RB, _D), lambda i: (i, 0)),
    out_shape=jax.ShapeDtypeStruct((_NPAD, _D), jnp.float32),
)


def _scale_body(hw_ref, hist_ref, g_ref, dinv_ref):
    dinv = _dinv_from_hist(hist_ref)
    g_ref[...] = hw_ref[...] * dinv
    dinv_ref[...] = jnp.broadcast_to(dinv, (_RB, _D))


_scale_call = pl.pallas_call(
    _scale_body,
    grid=(_NBLK,),
    in_specs=[
        pl.BlockSpec((_RB, _D), lambda i: (i, 0)),
        pl.BlockSpec((_NC, _RB, _D), lambda i: (0, i, 0)),
    ],
    out_specs=[
        pl.BlockSpec((_RB, _D), lambda i: (i, 0)),
        pl.BlockSpec((_RB, _D), lambda i: (i, 0)),
    ],
    out_shape=[
        jax.ShapeDtypeStruct((_NPAD, _D), jnp.float32),
        jax.ShapeDtypeStruct((_NPAD, _D), jnp.float32),
    ],
)


def _layer_body(acc_ref, g_ref, dinv_ref, b_ref, w_ref, o_ref):
    dinv = dinv_ref[...]
    h = jnp.maximum(
        dinv * (acc_ref[0] + acc_ref[1] + g_ref[...]) + b_ref[...], 0.0)
    o_ref[...] = jnp.dot(
        h, w_ref[...], preferred_element_type=jnp.float32) * dinv


_layer_call = pl.pallas_call(
    _layer_body,
    grid=(_NBLK,),
    in_specs=[
        pl.BlockSpec((_NC, _RB, _D), lambda i: (0, i, 0)),
        pl.BlockSpec((_RB, _D), lambda i: (i, 0)),
        pl.BlockSpec((_RB, _D), lambda i: (i, 0)),
        pl.BlockSpec((1, _D), lambda i: (0, 0)),
        pl.BlockSpec((_D, _D), lambda i: (0, 0)),
    ],
    out_specs=pl.BlockSpec((_RB, _D), lambda i: (i, 0)),
    out_shape=jax.ShapeDtypeStruct((_NPAD, _D), jnp.float32),
)


def _final_body(acc_ref, g_ref, dinv_ref, b_ref, batch_ref,
                pw1_ref, pb1_ref, pw2_ref, pb2_ref, z_ref,
                sums_ref, cnt_ref):
    i = pl.program_id(0)

    @pl.when(i == 0)
    def _():
        sums_ref[...] = jnp.zeros_like(sums_ref)
        cnt_ref[...] = jnp.zeros_like(cnt_ref)

    dinv = dinv_ref[...]
    h = jnp.maximum(
        dinv * (acc_ref[0] + acc_ref[1] + g_ref[...]) + b_ref[...], 0.0)
    bvec = batch_ref[0, 0, :]
    one = (lax.broadcasted_iota(jnp.int32, (_G, _RB), 0)
           == bvec[None, :]).astype(jnp.float32)
    sums_ref[...] += jnp.dot(one, h, preferred_element_type=jnp.float32)
    cnt_ref[...] += jnp.sum(one, axis=1, keepdims=True)

    @pl.when(i == _NBLK - 1)
    def _():
        hg = sums_ref[...] / jnp.maximum(cnt_ref[...], 1.0)
        t = jnp.maximum(
            jnp.dot(hg, pw1_ref[...], preferred_element_type=jnp.float32)
            + pb1_ref[...], 0.0)
        z = jnp.dot(
            t, pw2_ref[...], preferred_element_type=jnp.float32) + pb2_ref[...]
        nrm = jnp.sqrt(jnp.sum(z * z, axis=1, keepdims=True))
        z_ref[...] = z / jnp.maximum(nrm, 1e-12)


_final_call = pl.pallas_call(
    _final_body,
    grid=(_NBLK,),
    in_specs=[
        pl.BlockSpec((_NC, _RB, _D), lambda i: (0, i, 0)),
        pl.BlockSpec((_RB, _D), lambda i: (i, 0)),
        pl.BlockSpec((_RB, _D), lambda i: (i, 0)),
        pl.BlockSpec((1, _D), lambda i: (0, 0)),
        pl.BlockSpec((1, 1, _RB), lambda i: (i, 0, 0)),
        pl.BlockSpec((_D, _D), lambda i: (0, 0)),
        pl.BlockSpec((1, _D), lambda i: (0, 0)),
        pl.BlockSpec((_D, _D), lambda i: (0, 0)),
        pl.BlockSpec((1, _D), lambda i: (0, 0)),
    ],
    out_specs=pl.BlockSpec((_G, _D), lambda i: (0, 0)),
    out_shape=jax.ShapeDtypeStruct((_G, _D), jnp.float32),
    scratch_shapes=[
        pltpu.VMEM((_G, _D), jnp.float32),
        pltpu.VMEM((_G, _D), jnp.float32),
    ],
)


@functools.cache
def _sc_kernels():
    mesh = plsc.VectorSubcoreMesh(
        core_axis_name="c", subcore_axis_name="s",
        num_cores=_NC, num_subcores=_NS)
    msg_kernel = pl.kernel(
        _msg_body_sc,
        out_type=jax.ShapeDtypeStruct((_NC, _NPAD, _D), jnp.float32),
        mesh=mesh,
        scratch_types=[
            pltpu.VMEM_SHARED((_NPAD, _D), jnp.float32),
            pltpu.VMEM((_CH, _D), jnp.float32),
            pltpu.VMEM((_CH, _D), jnp.float32),
            pltpu.VMEM((_CPW // 2, _CH), jnp.int32),
            pltpu.VMEM((_CPW, _CH), jnp.int32),
            pltpu.SemaphoreType.DMA,
            pltpu.SemaphoreType.DMA,
            pltpu.SemaphoreType.DMA,
            pltpu.SemaphoreType.DMA,
            pltpu.SemaphoreType.DMA,
        ],
    )
    deg_kernel = pl.kernel(
        _deg_body_sc,
        out_type=jax.ShapeDtypeStruct((_NC, _NPAD, _D), jnp.float32),
        mesh=mesh,
        scratch_types=[
            pltpu.VMEM_SHARED((_NPAD, _D), jnp.float32),
            pltpu.VMEM((_CH, _D), jnp.float32),
            pltpu.VMEM((_CH, _D), jnp.float32),
            pltpu.VMEM((_CPW, _CH), jnp.int32),
            pltpu.SemaphoreType.DMA,
            pltpu.SemaphoreType.DMA,
            pltpu.SemaphoreType.DMA,
            pltpu.SemaphoreType.DMA,
            pltpu.SemaphoreType.DMA,
        ],
    )
    return msg_kernel, deg_kernel


def _deg_call(dst2):
    return _sc_kernels()[1](
        dst2,
        jnp.ones((_CH, _D), jnp.float32),
        jnp.zeros((_CH, _D), jnp.float32),
    )


def _msg_call(g, src2, dst2):
    return _sc_kernels()[0](g, src2, dst2, jnp.zeros((_CH, _D), jnp.float32))


def kernel(x, edge_index, batch, W1, b1, W2, b2, PW1, Pb1, PW2, Pb2):
    f32 = jnp.float32
    i32 = jnp.int32
    npd = _EPAD - _E
    pad_src = (jnp.arange(npd, dtype=i32) * 13) % _N
    pad_dst = _N + jnp.arange(npd, dtype=i32) % (_NPAD - _N)
    src2 = jnp.concatenate(
        [edge_index[0], pad_src]).reshape(_EPAD // _CH, _CH)
    dst2 = jnp.concatenate(
        [edge_index[1], pad_dst]).reshape(_EPAD // _CH, _CH)
    xp = jnp.zeros((_NPAD, _D), f32).at[:_N].set(x)
    batch_p = jnp.concatenate(
        [batch.astype(i32), jnp.full((_NPAD - _N,), _G, i32)]
    ).reshape(_NBLK, 1, _RB)

    hw1 = _hw_call(xp, W1)
    hist = _deg_call(dst2)
    g1, dinvb = _scale_call(hw1, hist)
    acc1 = _msg_call(g1, src2, dst2)
    g2 = _layer_call(acc1, g1, dinvb, b1.reshape(1, _D), W2)
    acc2 = _msg_call(g2, src2, dst2)
    z = _final_call(acc2, g2, dinvb, b2.reshape(1, _D), batch_p,
                    PW1, Pb1.reshape(1, _D), PW2, Pb2.reshape(1, _D))
    return z


# speedup vs baseline: 1.0031x; 1.0031x over previous
import functools

import jax
import jax.numpy as jnp
from jax import lax
from jax.experimental import pallas as pl
from jax.experimental.pallas import tpu as pltpu
from jax.experimental.pallas import tpu_sc as plsc

_N = 10000
_E = 320000
_D = 128
_G = 16
_NPAD = 10240
_NC = 2
_NS = 16
_NW = _NC * _NS
_CH = 128
_CPW = 80
_EW = _CPW * _CH
_EPAD = _EW * _NW
_RPT = _NPAD // _NS
_RB = 512
_NBLK = _NPAD // _RB

def _msg_body_sc(g_hbm, src2_hbm, dst2_hbm, zeros_hbm, acc_out,
                 acc_sh, rows0, rows1, srcb, dstb,
                 semg0, semg1, sems0, sems1, semi):
    cx = lax.axis_index("c")
    sx = lax.axis_index("s")
    wid = sx * _NC + cx
    r0 = wid * _CPW
    row0 = sx * _RPT
    cp_src = pltpu.async_copy(src2_hbm.at[pl.ds(r0, _CPW // 2)], srcb, semi)
    cp_dst = pltpu.async_copy(dst2_hbm.at[pl.ds(r0, _CPW)], dstb, semi)
    pltpu.sync_copy(zeros_hbm, rows0)
    for j in range(_RPT // _CH):
        pltpu.sync_copy(rows0, acc_sh.at[pl.ds(row0 + j * _CH, _CH)])
    plsc.subcore_barrier()
    cp_src.wait()
    cp_dst.wait()

    rows = (rows0, rows1)
    semg = (semg0, semg1)
    sems = (sems0, sems1)
    cp_g = [None] * _CPW
    cp_s = [None] * _CPW
    half = _CPW // 2
    g_waited = [False] * _CPW
    for c in range(_CPW):
        if c == half:
            for d in (half - 1,):
                if not g_waited[d]:
                    cp_g[d].wait()
                    g_waited[d] = True
            pltpu.sync_copy(src2_hbm.at[pl.ds(r0 + half, half)], srcb)
        if c >= 2:
            cp_s[c - 2].wait()
        cb = c % 2
        cp_g[c] = pltpu.async_copy(
            g_hbm.at[srcb.at[c % half]], rows[cb], semg[cb])
        if c >= 1:
            d = c - 1
            if not g_waited[d]:
                cp_g[d].wait()
                g_waited[d] = True
            cp_s[d] = pltpu.async_copy(
                rows[d % 2], acc_sh.at[dstb.at[d]], sems[d % 2], add=True)
    d = _CPW - 1
    if not g_waited[d]:
        cp_g[d].wait()
        g_waited[d] = True
    cp_s[d] = pltpu.async_copy(
        rows[d % 2], acc_sh.at[dstb.at[d]], sems[d % 2], add=True)
    for d in range(_CPW - 2, _CPW):
        cp_s[d].wait()
    plsc.subcore_barrier()
    pltpu.sync_copy(acc_sh.at[pl.ds(row0, _RPT)],
                    acc_out.at[cx, pl.ds(row0, _RPT)])


def _deg_body_sc(dst2_hbm, ones_hbm, zeros_hbm, acc_out,
                 acc_sh, ones_v, z_v, dstb, sem0, sem1, sem2, sem3, semi):
    cx = lax.axis_index("c")
    sx = lax.axis_index("s")
    wid = sx * _NC + cx
    r0 = wid * _CPW
    row0 = sx * _RPT
    cp_dst = pltpu.async_copy(dst2_hbm.at[pl.ds(r0, _CPW)], dstb, semi)
    pltpu.sync_copy(ones_hbm, ones_v)
    pltpu.sync_copy(zeros_hbm, z_v)
    for j in range(_RPT // _CH):
        pltpu.sync_copy(z_v, acc_sh.at[pl.ds(row0 + j * _CH, _CH)])
    plsc.subcore_barrier()
    cp_dst.wait()
    sems = (sem0, sem1, sem2, sem3)
    cp_s = [None] * _CPW
    for c in range(_CPW):
        if c >= 4:
            cp_s[c - 4].wait()
        cp_s[c] = pltpu.async_copy(
            ones_v, acc_sh.at[dstb.at[c]], sems[c % 4], add=True)
    for c in range(_CPW - 4, _CPW):
        cp_s[c].wait()
    plsc.subcore_barrier()
    pltpu.sync_copy(acc_sh.at[pl.ds(row0, _RPT)],
                    acc_out.at[cx, pl.ds(row0, _RPT)])


def _dinv_from_hist(hist_ref):
    deg = hist_ref[0, :, 0:1] + hist_ref[1, :, 0:1] + 1.0
    return lax.rsqrt(deg)


def _g1_body(x_ref, w_ref, hist_ref, g_ref, dinv_ref):
    dinv = _dinv_from_hist(hist_ref)
    g_ref[...] = jnp.dot(
        x_ref[...], w_ref[...], preferred_element_type=jnp.float32) * dinv
    dinv_ref[...] = jnp.broadcast_to(dinv, (_RB, _D))


_g1_call = pl.pallas_call(
    _g1_body,
    grid=(_NBLK,),
    in_specs=[
        pl.BlockSpec((_RB, _D), lambda i: (i, 0)),
        pl.BlockSpec((_D, _D), lambda i: (0, 0)),
        pl.BlockSpec((_NC, _RB, _D), lambda i: (0, i, 0)),
    ],
    out_specs=[
        pl.BlockSpec((_RB, _D), lambda i: (i, 0)),
        pl.BlockSpec((_RB, _D), lambda i: (i, 0)),
    ],
    out_shape=[
        jax.ShapeDtypeStruct((_NPAD, _D), jnp.float32),
        jax.ShapeDtypeStruct((_NPAD, _D), jnp.float32),
    ],
)


def _layer_body(acc_ref, g_ref, dinv_ref, b_ref, w_ref, o_ref):
    dinv = dinv_ref[...]
    h = jnp.maximum(
        dinv * (acc_ref[0] + acc_ref[1] + g_ref[...]) + b_ref[...], 0.0)
    o_ref[...] = jnp.dot(
        h, w_ref[...], preferred_element_type=jnp.float32) * dinv


_layer_call = pl.pallas_call(
    _layer_body,
    grid=(_NBLK,),
    in_specs=[
        pl.BlockSpec((_NC, _RB, _D), lambda i: (0, i, 0)),
        pl.BlockSpec((_RB, _D), lambda i: (i, 0)),
        pl.BlockSpec((_RB, _D), lambda i: (i, 0)),
        pl.BlockSpec((1, _D), lambda i: (0, 0)),
        pl.BlockSpec((_D, _D), lambda i: (0, 0)),
    ],
    out_specs=pl.BlockSpec((_RB, _D), lambda i: (i, 0)),
    out_shape=jax.ShapeDtypeStruct((_NPAD, _D), jnp.float32),
)


def _final_body(acc_ref, g_ref, dinv_ref, b_ref, batch_ref,
                pw1_ref, pb1_ref, pw2_ref, pb2_ref, z_ref,
                sums_ref, cnt_ref):
    i = pl.program_id(0)

    @pl.when(i == 0)
    def _():
        sums_ref[...] = jnp.zeros_like(sums_ref)
        cnt_ref[...] = jnp.zeros_like(cnt_ref)

    dinv = dinv_ref[...]
    h = jnp.maximum(
        dinv * (acc_ref[0] + acc_ref[1] + g_ref[...]) + b_ref[...], 0.0)
    bvec = batch_ref[0, 0, :]
    one = (lax.broadcasted_iota(jnp.int32, (_G, _RB), 0)
           == bvec[None, :]).astype(jnp.float32)
    sums_ref[...] += jnp.dot(one, h, preferred_element_type=jnp.float32)
    cnt_ref[...] += jnp.sum(one, axis=1, keepdims=True)

    @pl.when(i == _NBLK - 1)
    def _():
        hg = sums_ref[...] / jnp.maximum(cnt_ref[...], 1.0)
        t = jnp.maximum(
            jnp.dot(hg, pw1_ref[...], preferred_element_type=jnp.float32)
            + pb1_ref[...], 0.0)
        z = jnp.dot(
            t, pw2_ref[...], preferred_element_type=jnp.float32) + pb2_ref[...]
        nrm = jnp.sqrt(jnp.sum(z * z, axis=1, keepdims=True))
        z_ref[...] = z / jnp.maximum(nrm, 1e-12)


_final_call = pl.pallas_call(
    _final_body,
    grid=(_NBLK,),
    in_specs=[
        pl.BlockSpec((_NC, _RB, _D), lambda i: (0, i, 0)),
        pl.BlockSpec((_RB, _D), lambda i: (i, 0)),
        pl.BlockSpec((_RB, _D), lambda i: (i, 0)),
        pl.BlockSpec((1, _D), lambda i: (0, 0)),
        pl.BlockSpec((1, 1, _RB), lambda i: (i, 0, 0)),
        pl.BlockSpec((_D, _D), lambda i: (0, 0)),
        pl.BlockSpec((1, _D), lambda i: (0, 0)),
        pl.BlockSpec((_D, _D), lambda i: (0, 0)),
        pl.BlockSpec((1, _D), lambda i: (0, 0)),
    ],
    out_specs=pl.BlockSpec((_G, _D), lambda i: (0, 0)),
    out_shape=jax.ShapeDtypeStruct((_G, _D), jnp.float32),
    scratch_shapes=[
        pltpu.VMEM((_G, _D), jnp.float32),
        pltpu.VMEM((_G, _D), jnp.float32),
    ],
)


@functools.cache
def _sc_kernels():
    mesh = plsc.VectorSubcoreMesh(
        core_axis_name="c", subcore_axis_name="s",
        num_cores=_NC, num_subcores=_NS)
    msg_kernel = pl.kernel(
        _msg_body_sc,
        out_type=jax.ShapeDtypeStruct((_NC, _NPAD, _D), jnp.float32),
        mesh=mesh,
        scratch_types=[
            pltpu.VMEM_SHARED((_NPAD, _D), jnp.float32),
            pltpu.VMEM((_CH, _D), jnp.float32),
            pltpu.VMEM((_CH, _D), jnp.float32),
            pltpu.VMEM((_CPW // 2, _CH), jnp.int32),
            pltpu.VMEM((_CPW, _CH), jnp.int32),
            pltpu.SemaphoreType.DMA,
            pltpu.SemaphoreType.DMA,
            pltpu.SemaphoreType.DMA,
            pltpu.SemaphoreType.DMA,
            pltpu.SemaphoreType.DMA,
        ],
    )
    deg_kernel = pl.kernel(
        _deg_body_sc,
        out_type=jax.ShapeDtypeStruct((_NC, _NPAD, _D), jnp.float32),
        mesh=mesh,
        scratch_types=[
            pltpu.VMEM_SHARED((_NPAD, _D), jnp.float32),
            pltpu.VMEM((_CH, _D), jnp.float32),
            pltpu.VMEM((_CH, _D), jnp.float32),
            pltpu.VMEM((_CPW, _CH), jnp.int32),
            pltpu.SemaphoreType.DMA,
            pltpu.SemaphoreType.DMA,
            pltpu.SemaphoreType.DMA,
            pltpu.SemaphoreType.DMA,
            pltpu.SemaphoreType.DMA,
        ],
    )
    return msg_kernel, deg_kernel


def _deg_call(dst2):
    return _sc_kernels()[1](
        dst2,
        jnp.ones((_CH, _D), jnp.float32),
        jnp.zeros((_CH, _D), jnp.float32),
    )


def _msg_call(g, src2, dst2):
    return _sc_kernels()[0](g, src2, dst2, jnp.zeros((_CH, _D), jnp.float32))


def kernel(x, edge_index, batch, W1, b1, W2, b2, PW1, Pb1, PW2, Pb2):
    f32 = jnp.float32
    i32 = jnp.int32
    npd = _EPAD - _E
    pad_src = (jnp.arange(npd, dtype=i32) * 13) % _N
    pad_dst = _N + jnp.arange(npd, dtype=i32) % (_NPAD - _N)
    src2 = jnp.concatenate(
        [edge_index[0], pad_src]).reshape(_EPAD // _CH, _CH)
    dst2 = jnp.concatenate(
        [edge_index[1], pad_dst]).reshape(_EPAD // _CH, _CH)
    xp = jnp.zeros((_NPAD, _D), f32).at[:_N].set(x)
    batch_p = jnp.concatenate(
        [batch.astype(i32), jnp.full((_NPAD - _N,), _G, i32)]
    ).reshape(_NBLK, 1, _RB)

    hist = _deg_call(dst2)
    g1, dinvb = _g1_call(xp, W1, hist)
    acc1 = _msg_call(g1, src2, dst2)
    g2 = _layer_call(acc1, g1, dinvb, b1.reshape(1, _D), W2)
    acc2 = _msg_call(g2, src2, dst2)
    z = _final_call(acc2, g2, dinvb, b2.reshape(1, _D), batch_p,
                    PW1, Pb1.reshape(1, _D), PW2, Pb2.reshape(1, _D))
    return z
